# baseline (speedup 1.0000x reference)
import functools

import jax
import jax.numpy as jnp
from jax import lax
from jax.experimental import pallas as pl
from jax.experimental.pallas import tpu as pltpu
from jax.experimental.pallas import tpu_sc as plsc

NC = 2
NS = 16
L = 16

V = 1_000_000
SUP = 512
NSUP = 999_936 // SUP
TAIL0 = NSUP * SUP
ITER = -(-NSUP // (NC * NS))


def _sc_repack(in_t, in_tail, out_t, out_tail):
    mesh = plsc.VectorSubcoreMesh(
        core_axis_name="c", subcore_axis_name="s",
        num_cores=NC, num_subcores=NS)

    @functools.partial(
        pl.kernel,
        out_type=(jax.ShapeDtypeStruct((V // 4, 128), jnp.float32),
                  jax.ShapeDtypeStruct((V // 4, 128), jnp.float32)),
        mesh=mesh,
        compiler_params=pltpu.CompilerParams(needs_layout_passes=False),
        scratch_types=[
            pltpu.VMEM((2, 32, SUP + 1), jnp.float32),
            pltpu.VMEM((2, SUP // 4, 128), jnp.float32),
            pltpu.VMEM((64, 32), jnp.float32),
            pltpu.VMEM((16, 128), jnp.float32),
            pltpu.SemaphoreType.DMA,
            pltpu.SemaphoreType.DMA,
            pltpu.SemaphoreType.DMA,
            pltpu.SemaphoreType.DMA,
        ],
    )
    def repack(in_t_h, in_tail_h, out_t_h, out_tail_h, in_p_h, out_p_h,
               inb, outb, tailv, tout, semi0, semi1, semo0, semo1):
        wid = lax.axis_index("s") * NC + lax.axis_index("c")
        i16 = lax.iota(jnp.int32, L)

        def run_table(src_h, tail_h, dst_h):
            semi = (semi0, semi1)
            semo = (semo0, semo1)

            def stage(i, par):
                s = i * 32 + wid

                @pl.when(s < NSUP)
                def _():
                    v0 = pl.multiple_of(s * SUP, SUP)
                    for dg in range(4):
                        pltpu.async_copy(
                            src_h.at[pl.ds(dg * 8, 8), pl.ds(v0, SUP)],
                            inb.at[par, pl.ds(dg * 8, 8), pl.ds(0, SUP)],
                            semi[par])

            def drain_in(par):
                for dg in range(4):
                    pltpu.make_async_copy(
                        src_h.at[pl.ds(0, 8), pl.ds(0, SUP)],
                        inb.at[par, pl.ds(dg * 8, 8), pl.ds(0, SUP)],
                        semi[par]).wait()

            def drain_out(par):
                pltpu.make_async_copy(
                    dst_h.at[pl.ds(0, SUP // 4)], outb.at[par],
                    semo[par]).wait()

            stage(0, 0)

            @pl.loop(0, ITER, step=2)
            def _blk(io):
                for par in range(2):
                    i = io + par
                    s = i * 32 + wid
                    stage(i + 1, 1 - par)

                    @pl.when((i >= 2) & ((i - 2) * 32 + wid < NSUP))
                    def _():
                        drain_out(par)

                    @pl.when(s < NSUP)
                    def _():
                        drain_in(par)

                        @pl.loop(0, SUP // 4, step=8)
                        def _tr(r0):
                            for dr in range(8):
                                r = r0 + dr
                                for p in range(4):
                                    vcol = jnp.full((L,), 0, jnp.int32) + (
                                        4 * r + p)
                                    for h in range(2):
                                        vec = plsc.load_gather(
                                            inb.at[par],
                                            [h * L + i16, vcol])
                                        outb[par, r,
                                             pl.ds(p * 32 + h * L, L)] = vec

                        pltpu.async_copy(
                            outb.at[par],
                            dst_h.at[pl.ds(
                                pl.multiple_of(s * (SUP // 4), SUP // 4),
                                SUP // 4)],
                            semo[par])

            for ii in (ITER - 2, ITER - 1):
                @pl.when(ii * 32 + wid < NSUP)
                def _():
                    drain_out(ii % 2)

            @pl.when(wid == 0)
            def _():
                pltpu.sync_copy(tail_h, tailv)
                for v in range(64):
                    for h in range(2):
                        tout[v >> 2, pl.ds((v & 3) * 32 + h * L, L)] = (
                            tailv[v, pl.ds(h * L, L)])
                pltpu.sync_copy(tout, dst_h.at[pl.ds(TAIL0 // 4, 16)])

        run_table(in_t_h, in_tail_h, in_p_h)
        run_table(out_t_h, out_tail_h, out_p_h)

    return repack(in_t, in_tail, out_t, out_tail)


def _sc_scores(center2d, context2d, neg2d, in_w, out_w, B, K, D):
    NW = NC * NS
    BPW = B // NW
    NEG_PW = BPW * K
    IDXW = 128
    CROWS = B // IDXW // NW
    NROWS = NEG_PW // IDXW
    BLK_B = 32
    RPB = BLK_B * K
    DPB = RPB // IDXW
    NBLK = BPW // BLK_B

    mesh = plsc.VectorSubcoreMesh(
        core_axis_name="c", subcore_axis_name="s",
        num_cores=NC, num_subcores=NS)

    @functools.partial(
        pl.kernel,
        out_type=(jax.ShapeDtypeStruct((B,), jnp.float32),
                  jax.ShapeDtypeStruct((B * K,), jnp.float32)),
        mesh=mesh,
        compiler_params=pltpu.CompilerParams(
            needs_layout_passes=False, use_tc_tiling_on_sc=False),
        scratch_types=[
            pltpu.VMEM((CROWS, IDXW), jnp.int32),
            pltpu.VMEM((CROWS, IDXW), jnp.int32),
            pltpu.VMEM((NROWS, IDXW), jnp.int32),
            pltpu.VMEM((BPW, D), jnp.float32),
            pltpu.VMEM((BPW, D), jnp.float32),
            pltpu.VMEM((2, RPB, D), jnp.float32),
            pltpu.VMEM((BPW,), jnp.float32),
            pltpu.VMEM((NEG_PW,), jnp.float32),
            pltpu.SemaphoreType.DMA,
            pltpu.SemaphoreType.DMA,
            pltpu.SemaphoreType.DMA,
        ],
    )
    def sc_kernel(in_w_h, out_w_h, c2_h, x2_h, n2_h, pos_h, neg_h,
                  cidx, xidx, nidx, vrows, prows, nbuf, posv, negv,
                  sem_vp, semn0, semn1):
        wid = lax.axis_index("s") * NC + lax.axis_index("c")

        pltpu.sync_copy(c2_h.at[pl.ds(wid * CROWS, CROWS)], cidx)
        pltpu.sync_copy(x2_h.at[pl.ds(wid * CROWS, CROWS)], xidx)
        pltpu.sync_copy(n2_h.at[pl.ds(wid * NROWS, NROWS)], nidx)

        vp_copies = []
        for j in range(CROWS):
            vp_copies.append(pltpu.async_copy(
                in_w_h.at[cidx.at[j]], vrows.at[pl.ds(j * IDXW, IDXW)], sem_vp))
            vp_copies.append(pltpu.async_copy(
                out_w_h.at[xidx.at[j]], prows.at[pl.ds(j * IDXW, IDXW)], sem_vp))

        def issue_neg(g, slot, sem):
            for j in range(DPB):
                pltpu.async_copy(
                    out_w_h.at[nidx.at[g * DPB + j]],
                    nbuf.at[slot, pl.ds(j * IDXW, IDXW)], sem)

        issue_neg(0, 0, semn0)
        issue_neg(1, 1, semn1)
        for c in vp_copies:
            c.wait()

        i16 = lax.iota(jnp.int32, L)
        i20 = i16 * K

        @pl.loop(0, NBLK, step=2)
        def _blk(go):
            for par in range(2):
                g = go + par
                nsem = semn0 if par == 0 else semn1
                pltpu.make_async_copy(
                    out_w_h.at[pl.ds(0, RPB)], nbuf.at[par], nsem).wait()
                for sb in range(2):
                    b0 = g * BLK_B + sb * L
                    bvec = b0 + i16
                    vcols = [plsc.load_gather(
                        vrows, [bvec, jnp.full((L,), d, jnp.int32)])
                        for d in range(D)]
                    pa = [jnp.zeros((L,), jnp.float32) for _ in range(4)]
                    for d in range(D):
                        u = plsc.load_gather(
                            prows, [bvec, jnp.full((L,), d, jnp.int32)])
                        pa[d % 4] = pa[d % 4] + u * vcols[d]
                    posv[pl.ds(b0, L)] = (pa[0] + pa[1]) + (pa[2] + pa[3])
                    rbase = sb * L * K
                    obase = b0 * K

                    @pl.loop(0, K)
                    def _k(k):
                        rv = rbase + i20 + k
                        na = [jnp.zeros((L,), jnp.float32) for _ in range(4)]
                        for d in range(D):
                            u = plsc.load_gather(
                                nbuf.at[par],
                                [rv, jnp.full((L,), d, jnp.int32)])
                            na[d % 4] = na[d % 4] + u * vcols[d]
                        plsc.store_scatter(
                            negv, [obase + i20 + k],
                            (na[0] + na[1]) + (na[2] + na[3]))

                @pl.when(g + 2 < NBLK)
                def _():
                    issue_neg(g + 2, par, nsem)

        pltpu.sync_copy(posv, pos_h.at[pl.ds(wid * BPW, BPW)])
        pltpu.sync_copy(negv, neg_h.at[pl.ds(wid * NEG_PW, NEG_PW)])

    return sc_kernel(in_w, out_w, center2d, context2d, neg2d)


def _loss_reduce(pos_score, neg_flat, B):
    def body(pos_ref, neg_ref, out_ref):
        def logsig(x):
            return jnp.minimum(x, 0.0) - jnp.log1p(jnp.exp(-jnp.abs(x)))
        s = jnp.sum(logsig(pos_ref[...])) + jnp.sum(logsig(-neg_ref[...]))
        out_ref[0, 0] = -s / B

    out = pl.pallas_call(
        body,
        out_shape=jax.ShapeDtypeStruct((1, 1), jnp.float32),
        out_specs=pl.BlockSpec(memory_space=pltpu.SMEM),
    )(pos_score.reshape(B // 128, 128),
      neg_flat.reshape(-1, 128))
    return out.reshape(())


def kernel(center, context, negative, in_embed_w, out_embed_w):
    B, = center.shape
    K = negative.shape[1]
    D = in_embed_w.shape[1]
    center2d = center.astype(jnp.int32).reshape(B // 128, 128)
    context2d = context.astype(jnp.int32).reshape(B // 128, 128)
    neg2d = negative.astype(jnp.int32).reshape(B * K // 128, 128)
    in_p, out_p = _sc_repack(
        in_embed_w.T, in_embed_w[TAIL0:, :],
        out_embed_w.T, out_embed_w[TAIL0:, :])
    in_p, out_p = jax.lax.optimization_barrier((in_p, out_p))
    in_rm = in_p.reshape(V, D)
    out_rm = out_p.reshape(V, D)
    pos_score, neg_flat = _sc_scores(
        center2d, context2d, neg2d, in_rm, out_rm, B, K, D)
    return _loss_reduce(pos_score, neg_flat, B)


# speedup vs baseline: 1.6485x; 1.6485x over previous
import functools

import jax
import jax.numpy as jnp
from jax import lax
from jax.experimental import pallas as pl
from jax.experimental.pallas import tpu as pltpu
from jax.experimental.pallas import tpu_sc as plsc

NC = 2
NS = 16
L = 16


def _sc_scores(center2d, context2d, neg2d, in_w, out_w, B, K, D):
    NW = NC * NS
    BPW = B // NW
    NEG_PW = BPW * K
    IDXW = 128
    CROWS = B // IDXW // NW
    NROWS = NEG_PW // IDXW
    BLK_B = 32
    RPB = BLK_B * K
    DPB = RPB // IDXW
    NBLK = BPW // BLK_B

    mesh = plsc.VectorSubcoreMesh(
        core_axis_name="c", subcore_axis_name="s",
        num_cores=NC, num_subcores=NS)

    @functools.partial(
        pl.kernel,
        out_type=(jax.ShapeDtypeStruct((B,), jnp.float32),
                  jax.ShapeDtypeStruct((B * K,), jnp.float32)),
        mesh=mesh,
        compiler_params=pltpu.CompilerParams(
            needs_layout_passes=False, use_tc_tiling_on_sc=False),
        scratch_types=[
            pltpu.VMEM((CROWS, IDXW), jnp.int32),
            pltpu.VMEM((CROWS, IDXW), jnp.int32),
            pltpu.VMEM((NROWS, IDXW), jnp.int32),
            pltpu.VMEM((BPW, D), jnp.float32),
            pltpu.VMEM((BPW, D), jnp.float32),
            pltpu.VMEM((2, RPB, D), jnp.float32),
            pltpu.VMEM((BPW,), jnp.float32),
            pltpu.VMEM((NEG_PW,), jnp.float32),
            pltpu.SemaphoreType.DMA,
            pltpu.SemaphoreType.DMA,
            pltpu.SemaphoreType.DMA,
        ],
    )
    def sc_kernel(in_w_h, out_w_h, c2_h, x2_h, n2_h, pos_h, neg_h,
                  cidx, xidx, nidx, vrows, prows, nbuf, posv, negv,
                  sem_vp, semn0, semn1):
        wid = lax.axis_index("s") * NC + lax.axis_index("c")

        pltpu.sync_copy(c2_h.at[pl.ds(wid * CROWS, CROWS)], cidx)
        pltpu.sync_copy(x2_h.at[pl.ds(wid * CROWS, CROWS)], xidx)
        pltpu.sync_copy(n2_h.at[pl.ds(wid * NROWS, NROWS)], nidx)

        vp_copies = []
        for j in range(CROWS):
            vp_copies.append(pltpu.async_copy(
                in_w_h.at[cidx.at[j]], vrows.at[pl.ds(j * IDXW, IDXW)], sem_vp))
            vp_copies.append(pltpu.async_copy(
                out_w_h.at[xidx.at[j]], prows.at[pl.ds(j * IDXW, IDXW)], sem_vp))

        def issue_neg(g, slot, sem):
            for j in range(DPB):
                pltpu.async_copy(
                    out_w_h.at[nidx.at[g * DPB + j]],
                    nbuf.at[slot, pl.ds(j * IDXW, IDXW)], sem)

        issue_neg(0, 0, semn0)
        issue_neg(1, 1, semn1)
        for c in vp_copies:
            c.wait()

        i16 = lax.iota(jnp.int32, L)
        i20 = i16 * K

        @pl.loop(0, NBLK, step=2)
        def _blk(go):
            for par in range(2):
                g = go + par
                nsem = semn0 if par == 0 else semn1
                pltpu.make_async_copy(
                    out_w_h.at[pl.ds(0, RPB)], nbuf.at[par], nsem).wait()
                for sb in range(2):
                    b0 = g * BLK_B + sb * L
                    bvec = b0 + i16
                    vcols = [plsc.load_gather(
                        vrows, [bvec, jnp.full((L,), d, jnp.int32)])
                        for d in range(D)]
                    pa = [jnp.zeros((L,), jnp.float32) for _ in range(4)]
                    for d in range(D):
                        u = plsc.load_gather(
                            prows, [bvec, jnp.full((L,), d, jnp.int32)])
                        pa[d % 4] = pa[d % 4] + u * vcols[d]
                    posv[pl.ds(b0, L)] = (pa[0] + pa[1]) + (pa[2] + pa[3])
                    rbase = sb * L * K
                    obase = b0 * K

                    @pl.loop(0, K)
                    def _k(k):
                        rv = rbase + i20 + k
                        na = [jnp.zeros((L,), jnp.float32) for _ in range(4)]
                        for d in range(D):
                            u = plsc.load_gather(
                                nbuf.at[par],
                                [rv, jnp.full((L,), d, jnp.int32)])
                            na[d % 4] = na[d % 4] + u * vcols[d]
                        plsc.store_scatter(
                            negv, [obase + i20 + k],
                            (na[0] + na[1]) + (na[2] + na[3]))

                @pl.when(g + 2 < NBLK)
                def _():
                    issue_neg(g + 2, par, nsem)

        pltpu.sync_copy(posv, pos_h.at[pl.ds(wid * BPW, BPW)])
        pltpu.sync_copy(negv, neg_h.at[pl.ds(wid * NEG_PW, NEG_PW)])

    return sc_kernel(in_w, out_w, center2d, context2d, neg2d)


def _loss_reduce(pos_score, neg_flat, B):
    def body(pos_ref, neg_ref, out_ref):
        def logsig(x):
            return jnp.minimum(x, 0.0) - jnp.log1p(jnp.exp(-jnp.abs(x)))
        s = jnp.sum(logsig(pos_ref[...])) + jnp.sum(logsig(-neg_ref[...]))
        out_ref[0, 0] = -s / B

    out = pl.pallas_call(
        body,
        out_shape=jax.ShapeDtypeStruct((1, 1), jnp.float32),
        out_specs=pl.BlockSpec(memory_space=pltpu.SMEM),
    )(pos_score.reshape(B // 128, 128),
      neg_flat.reshape(-1, 128))
    return out.reshape(())


def kernel(center, context, negative, in_embed_w, out_embed_w):
    B, = center.shape
    K = negative.shape[1]
    D = in_embed_w.shape[1]
    center2d = center.astype(jnp.int32).reshape(B // 128, 128)
    context2d = context.astype(jnp.int32).reshape(B // 128, 128)
    neg2d = negative.astype(jnp.int32).reshape(B * K // 128, 128)
    pos_score, neg_flat = _sc_scores(
        center2d, context2d, neg2d, in_embed_w, out_embed_w, B, K, D)
    return _loss_reduce(pos_score, neg_flat, B)
